# Initial kernel scaffold; baseline (speedup 1.0000x reference)
import jax
import jax.numpy as jnp
from jax.experimental import pallas as pl


def kernel(x, w0, b0, w1, b1, w2, b2):
    raise NotImplementedError("write your pallas kernel here")


# speedup vs baseline: 1.2794x; 1.2794x over previous
import functools

import jax
import jax.numpy as jnp
from jax.experimental import pallas as pl
from jax.experimental.pallas import tpu as pltpu


def _round_up(x: int, m: int) -> int:
    return (x + m - 1) // m * m


def _mlp_kernel(x_ref, w0_ref, b0_ref, w1_ref, b1_ref, w2_ref, b2_ref, o_ref):
    x = x_ref[...].astype(jnp.bfloat16)
    h = jnp.dot(x, w0_ref[...], preferred_element_type=jnp.float32)
    h = jnp.maximum(h + b0_ref[...], 0.0).astype(jnp.bfloat16)
    h = jnp.dot(h, w1_ref[...], preferred_element_type=jnp.float32)
    h = jnp.maximum(h + b1_ref[...], 0.0).astype(jnp.bfloat16)
    h = jnp.dot(h, w2_ref[...], preferred_element_type=jnp.float32)
    o_ref[...] = (h + b2_ref[...]).astype(o_ref.dtype)


def _pick_tm(m: int) -> int:
    for tm in (576, 512, 384, 320, 256):
        if m % tm == 0:
            return tm
    return 256


def kernel(x, w0, b0, w1, b1, w2, b2):
    lead_shape = x.shape[:-1]
    m = 1
    for d in lead_shape:
        m *= d
    din = x.shape[-1]
    dout = w2.shape[1]

    xf = x.reshape(m, din)
    tm = _pick_tm(m)
    m_pad = _round_up(m, tm)
    if m_pad != m:
        xf = jnp.zeros((m_pad, din), xf.dtype).at[:m].set(xf)

    w0b = w0.astype(jnp.bfloat16)
    w1b = w1.astype(jnp.bfloat16)
    w2b = w2.astype(jnp.bfloat16)

    grid = (m_pad // tm,)
    in_specs = [
        pl.BlockSpec((tm, din), lambda i: (i, 0)),
        pl.BlockSpec(w0b.shape, lambda i: (0, 0)),
        pl.BlockSpec(b0.shape, lambda i: (0, 0)),
        pl.BlockSpec(w1b.shape, lambda i: (0, 0)),
        pl.BlockSpec(b1.shape, lambda i: (0, 0)),
        pl.BlockSpec(w2b.shape, lambda i: (0, 0)),
        pl.BlockSpec(b2.shape, lambda i: (0, 0)),
    ]
    flops = 2 * m_pad * (w0.shape[0] * w0.shape[1]
                         + w1.shape[0] * w1.shape[1]
                         + w2.shape[0] * w2.shape[1])
    bytes_accessed = (xf.size * 4 + m_pad * dout * 4
                      + 2 * (w0.size + w1.size + w2.size)
                      + 4 * (b0.size + b1.size + b2.size))

    out_pad = pl.pallas_call(
        _mlp_kernel,
        out_shape=jax.ShapeDtypeStruct((m_pad, dout), x.dtype),
        grid=grid,
        in_specs=in_specs,
        out_specs=pl.BlockSpec((tm, dout), lambda i: (i, 0)),
        compiler_params=pltpu.CompilerParams(
            dimension_semantics=("parallel",),
        ),
        cost_estimate=pl.CostEstimate(
            flops=int(flops), transcendentals=0, bytes_accessed=int(bytes_accessed)
        ),
    )(xf, w0b, b0, w1b, b1, w2b, b2)

    return out_pad[:m].reshape(*lead_shape, dout)


# speedup vs baseline: 2.1342x; 1.6681x over previous
import jax
import jax.numpy as jnp
from jax.experimental import pallas as pl
from jax.experimental.pallas import tpu as pltpu


def _mlp_kernel(x_ref, w0_ref, b0_ref, w1_ref, b1_ref, w2_ref, b2_ref, o_ref):
    x = x_ref[0].astype(jnp.bfloat16)
    h = jnp.dot(x, w0_ref[...], preferred_element_type=jnp.float32)
    h = jnp.maximum(h + b0_ref[...], 0.0).astype(jnp.bfloat16)
    h = jnp.dot(h, w1_ref[...], preferred_element_type=jnp.float32)
    h = jnp.maximum(h + b1_ref[...], 0.0).astype(jnp.bfloat16)
    h = jnp.dot(h, w2_ref[...], preferred_element_type=jnp.float32)
    o_ref[0] = (h + b2_ref[...]).astype(o_ref.dtype)


def kernel(x, w0, b0, w1, b1, w2, b2):
    batch, rows, din = x.shape
    dout = w2.shape[1]

    w0b = w0.astype(jnp.bfloat16)
    w1b = w1.astype(jnp.bfloat16)
    w2b = w2.astype(jnp.bfloat16)

    grid = (batch,)
    in_specs = [
        pl.BlockSpec((1, rows, din), lambda i: (i, 0, 0)),
        pl.BlockSpec(w0b.shape, lambda i: (0, 0)),
        pl.BlockSpec(b0.shape, lambda i: (0, 0)),
        pl.BlockSpec(w1b.shape, lambda i: (0, 0)),
        pl.BlockSpec(b1.shape, lambda i: (0, 0)),
        pl.BlockSpec(w2b.shape, lambda i: (0, 0)),
        pl.BlockSpec(b2.shape, lambda i: (0, 0)),
    ]
    m = batch * rows
    flops = 2 * m * (w0.shape[0] * w0.shape[1]
                     + w1.shape[0] * w1.shape[1]
                     + w2.shape[0] * w2.shape[1])
    bytes_accessed = (x.size * 4 + m * dout * 4
                      + 2 * (w0.size + w1.size + w2.size)
                      + 4 * (b0.size + b1.size + b2.size))

    return pl.pallas_call(
        _mlp_kernel,
        out_shape=jax.ShapeDtypeStruct((batch, rows, dout), x.dtype),
        grid=grid,
        in_specs=in_specs,
        out_specs=pl.BlockSpec((1, rows, dout), lambda i: (i, 0, 0)),
        compiler_params=pltpu.CompilerParams(
            dimension_semantics=("parallel",),
        ),
        cost_estimate=pl.CostEstimate(
            flops=int(flops), transcendentals=0, bytes_accessed=int(bytes_accessed)
        ),
    )(x, w0b, b0, w1b, b1, w2b, b2)


# speedup vs baseline: 2.4556x; 1.1506x over previous
import functools

import jax
import jax.numpy as jnp
from jax.experimental import pallas as pl
from jax.experimental.pallas import tpu as pltpu

_BB = 4


def _mlp_kernel(x_ref, w0_ref, b0_ref, w1_ref, b1_ref, w2_ref, b2_ref, o_ref,
                *, bb: int):
    for j in range(bb):
        x = x_ref[j].astype(jnp.bfloat16)
        h = jnp.dot(x, w0_ref[...], preferred_element_type=jnp.float32)
        h = jnp.maximum(h + b0_ref[...], 0.0).astype(jnp.bfloat16)
        h = jnp.dot(h, w1_ref[...], preferred_element_type=jnp.float32)
        h = jnp.maximum(h + b1_ref[...], 0.0).astype(jnp.bfloat16)
        h = jnp.dot(h, w2_ref[...], preferred_element_type=jnp.float32)
        o_ref[j] = (h + b2_ref[...]).astype(o_ref.dtype)


def kernel(x, w0, b0, w1, b1, w2, b2):
    batch, rows, din = x.shape
    dout = w2.shape[1]
    bb = _BB if batch % _BB == 0 else 1

    w0b = w0.astype(jnp.bfloat16)
    w1b = w1.astype(jnp.bfloat16)
    w2b = w2.astype(jnp.bfloat16)

    grid = (batch // bb,)
    in_specs = [
        pl.BlockSpec((bb, rows, din), lambda i: (i, 0, 0)),
        pl.BlockSpec(w0b.shape, lambda i: (0, 0)),
        pl.BlockSpec(b0.shape, lambda i: (0, 0)),
        pl.BlockSpec(w1b.shape, lambda i: (0, 0)),
        pl.BlockSpec(b1.shape, lambda i: (0, 0)),
        pl.BlockSpec(w2b.shape, lambda i: (0, 0)),
        pl.BlockSpec(b2.shape, lambda i: (0, 0)),
    ]
    m = batch * rows
    flops = 2 * m * (w0.shape[0] * w0.shape[1]
                     + w1.shape[0] * w1.shape[1]
                     + w2.shape[0] * w2.shape[1])
    bytes_accessed = (x.size * 4 + m * dout * 4
                      + 2 * (w0.size + w1.size + w2.size)
                      + 4 * (b0.size + b1.size + b2.size))

    return pl.pallas_call(
        functools.partial(_mlp_kernel, bb=bb),
        out_shape=jax.ShapeDtypeStruct((batch, rows, dout), x.dtype),
        grid=grid,
        in_specs=in_specs,
        out_specs=pl.BlockSpec((bb, rows, dout), lambda i: (i, 0, 0)),
        compiler_params=pltpu.CompilerParams(
            dimension_semantics=("parallel",),
        ),
        cost_estimate=pl.CostEstimate(
            flops=int(flops), transcendentals=0, bytes_accessed=int(bytes_accessed)
        ),
    )(x, w0b, b0, w1b, b1, w2b, b2)


# speedup vs baseline: 2.4776x; 1.0090x over previous
import functools

import jax
import jax.numpy as jnp
from jax.experimental import pallas as pl
from jax.experimental.pallas import tpu as pltpu

_BB = 8


def _mlp_kernel(x_ref, w0_ref, b0_ref, w1_ref, b1_ref, w2_ref, b2_ref, o_ref,
                *, bb: int):
    for j in range(bb):
        x = x_ref[j].astype(jnp.bfloat16)
        h = jnp.dot(x, w0_ref[...], preferred_element_type=jnp.float32)
        h = jnp.maximum(h + b0_ref[...], 0.0).astype(jnp.bfloat16)
        h = jnp.dot(h, w1_ref[...], preferred_element_type=jnp.float32)
        h = jnp.maximum(h + b1_ref[...], 0.0).astype(jnp.bfloat16)
        h = jnp.dot(h, w2_ref[...], preferred_element_type=jnp.float32)
        o_ref[j] = (h + b2_ref[...]).astype(o_ref.dtype)


def kernel(x, w0, b0, w1, b1, w2, b2):
    batch, rows, din = x.shape
    dout = w2.shape[1]
    bb = _BB if batch % _BB == 0 else 1

    w0b = w0.astype(jnp.bfloat16)
    w1b = w1.astype(jnp.bfloat16)
    w2b = w2.astype(jnp.bfloat16)

    grid = (batch // bb,)
    in_specs = [
        pl.BlockSpec((bb, rows, din), lambda i: (i, 0, 0)),
        pl.BlockSpec(w0b.shape, lambda i: (0, 0)),
        pl.BlockSpec(b0.shape, lambda i: (0, 0)),
        pl.BlockSpec(w1b.shape, lambda i: (0, 0)),
        pl.BlockSpec(b1.shape, lambda i: (0, 0)),
        pl.BlockSpec(w2b.shape, lambda i: (0, 0)),
        pl.BlockSpec(b2.shape, lambda i: (0, 0)),
    ]
    m = batch * rows
    flops = 2 * m * (w0.shape[0] * w0.shape[1]
                     + w1.shape[0] * w1.shape[1]
                     + w2.shape[0] * w2.shape[1])
    bytes_accessed = (x.size * 4 + m * dout * 4
                      + 2 * (w0.size + w1.size + w2.size)
                      + 4 * (b0.size + b1.size + b2.size))

    return pl.pallas_call(
        functools.partial(_mlp_kernel, bb=bb),
        out_shape=jax.ShapeDtypeStruct((batch, rows, dout), x.dtype),
        grid=grid,
        in_specs=in_specs,
        out_specs=pl.BlockSpec((bb, rows, dout), lambda i: (i, 0, 0)),
        compiler_params=pltpu.CompilerParams(
            dimension_semantics=("parallel",),
        ),
        cost_estimate=pl.CostEstimate(
            flops=int(flops), transcendentals=0, bytes_accessed=int(bytes_accessed)
        ),
    )(x, w0b, b0, w1b, b1, w2b, b2)


# speedup vs baseline: 3.8221x; 1.5427x over previous
import jax
import jax.numpy as jnp
from jax.experimental import pallas as pl
from jax.experimental.pallas import tpu as pltpu


def _mlp_kernel(x_ref, w0_ref, b0_ref, w1_ref, b1_ref, w2_ref, b2_ref, o_ref):
    x = x_ref[...].astype(jnp.bfloat16)
    h = jnp.dot(x, w0_ref[...], preferred_element_type=jnp.float32)
    h = jnp.maximum(h + b0_ref[...], 0.0).astype(jnp.bfloat16)
    h = jnp.dot(h, w1_ref[...], preferred_element_type=jnp.float32)
    h = jnp.maximum(h + b1_ref[...], 0.0).astype(jnp.bfloat16)
    h = jnp.dot(h, w2_ref[...], preferred_element_type=jnp.float32)
    o_ref[...] = (h + b2_ref[...]).astype(o_ref.dtype)


def _pick_tm(m: int) -> int:
    for tm in (1152, 1024, 960, 768, 640, 576, 512, 384, 320, 256):
        if m % tm == 0:
            return tm
    return 256


def kernel(x, w0, b0, w1, b1, w2, b2):
    batch, rows, din = x.shape
    dout = w2.shape[1]
    m = batch * rows

    xf = jnp.transpose(x, (1, 0, 2)).reshape(m, din)

    tm = _pick_tm(m)
    m_pad = (m + tm - 1) // tm * tm
    if m_pad != m:
        xf = jnp.zeros((m_pad, din), xf.dtype).at[:m].set(xf)

    w0b = w0.astype(jnp.bfloat16)
    w1b = w1.astype(jnp.bfloat16)
    w2b = w2.astype(jnp.bfloat16)

    grid = (m_pad // tm,)
    in_specs = [
        pl.BlockSpec((tm, din), lambda i: (i, 0)),
        pl.BlockSpec(w0b.shape, lambda i: (0, 0)),
        pl.BlockSpec(b0.shape, lambda i: (0, 0)),
        pl.BlockSpec(w1b.shape, lambda i: (0, 0)),
        pl.BlockSpec(b1.shape, lambda i: (0, 0)),
        pl.BlockSpec(w2b.shape, lambda i: (0, 0)),
        pl.BlockSpec(b2.shape, lambda i: (0, 0)),
    ]
    flops = 2 * m_pad * (w0.shape[0] * w0.shape[1]
                         + w1.shape[0] * w1.shape[1]
                         + w2.shape[0] * w2.shape[1])
    bytes_accessed = (xf.size * 4 + m_pad * dout * 4
                      + 2 * (w0.size + w1.size + w2.size)
                      + 4 * (b0.size + b1.size + b2.size))

    out_flat = pl.pallas_call(
        _mlp_kernel,
        out_shape=jax.ShapeDtypeStruct((m_pad, dout), x.dtype),
        grid=grid,
        in_specs=in_specs,
        out_specs=pl.BlockSpec((tm, dout), lambda i: (i, 0)),
        compiler_params=pltpu.CompilerParams(
            dimension_semantics=("parallel",),
        ),
        cost_estimate=pl.CostEstimate(
            flops=int(flops), transcendentals=0, bytes_accessed=int(bytes_accessed)
        ),
    )(xf, w0b, b0, w1b, b1, w2b, b2)

    return out_flat[:m].reshape(rows, batch, dout).transpose(1, 0, 2)
